# baseline (speedup 1.0000x reference)
import functools

import jax
import jax.numpy as jnp
from jax import lax
from jax.experimental import pallas as pl
from jax.experimental.pallas import tpu as pltpu
from jax.experimental.pallas import tpu_sc as plsc

NUM_NODES = 1024
DIM = 64
ROWS_PER_BLOCK = 8


def _scores_body(w_ref, c_ref, s_ref):
    w = w_ref[...]
    c = c_ref[0, :]
    e = jnp.sum(w * c[None, None, :], axis=-1)
    m = jnp.max(e, axis=-1, keepdims=True)
    p = jnp.exp(e - m)
    s_ref[...] = p / jnp.sum(p, axis=-1, keepdims=True)


def _compute_scores(W, c2):
    G = ROWS_PER_BLOCK
    N = NUM_NODES
    return pl.pallas_call(
        _scores_body,
        grid=(N // G,),
        in_specs=[
            pl.BlockSpec((G, N, DIM), lambda i: (i, 0, 0)),
            pl.BlockSpec((1, DIM), lambda i: (0, 0)),
        ],
        out_specs=pl.BlockSpec((G, N), lambda i: (i, 0)),
        out_shape=jax.ShapeDtypeStruct((N, N), jnp.float32),
    )(W, c2)


def _make_sc_gather(B, D):
    info = plsc.get_sparse_core_info()
    NC, NS = info.num_cores, info.num_subcores
    NW = NC * NS
    b_per_w = B // NW
    mesh = plsc.VectorSubcoreMesh(core_axis_name="c", subcore_axis_name="s")

    @functools.partial(
        pl.kernel,
        mesh=mesh,
        out_type=jax.ShapeDtypeStruct((B, D), jnp.float32),
        scratch_types=[
            pltpu.VMEM((b_per_w,), jnp.int32),
            pltpu.VMEM((b_per_w, D), jnp.float32),
            pltpu.SemaphoreType.DMA,
        ],
    )
    def gather_k(table_hbm, idx_hbm, out_hbm, idx_v, rows_v, sem):
        wid = lax.axis_index("s") * NC + lax.axis_index("c")
        base = wid * b_per_w
        pltpu.sync_copy(idx_hbm.at[pl.ds(base, b_per_w)], idx_v)
        pltpu.async_copy(table_hbm.at[idx_v], rows_v, sem).wait()
        pltpu.sync_copy(rows_v, out_hbm.at[pl.ds(base, b_per_w)])

    return gather_k


def kernel(node_indices, context_vector, W):
    c2 = context_vector.reshape(1, DIM)
    scores = _compute_scores(W, c2)
    gather = _make_sc_gather(node_indices.shape[0], NUM_NODES)
    return gather(scores, node_indices.astype(jnp.int32))


# speedup vs baseline: 1.0898x; 1.0898x over previous
import functools

import jax
import jax.numpy as jnp
from jax import lax
from jax.experimental import pallas as pl
from jax.experimental.pallas import tpu as pltpu
from jax.experimental.pallas import tpu_sc as plsc

NUM_NODES = 1024
DIM = 64
PAIRS = 8
FLAT_K = PAIRS * DIM
FLAT_ROWS = NUM_NODES * NUM_NODES // PAIRS
ROWS_PER_M = NUM_NODES // PAIRS
BLOCK_ROWS = 4096
M_PER_BLOCK = BLOCK_ROWS // ROWS_PER_M


def _scores_body(w_ref, b_ref, s_ref):
    a = w_ref[...].astype(jnp.bfloat16)
    e = jnp.dot(a, b_ref[...], preferred_element_type=jnp.float32)
    e3 = e.reshape(M_PER_BLOCK, ROWS_PER_M, PAIRS)
    m = jnp.max(jnp.max(e3, axis=2, keepdims=True), axis=1, keepdims=True)
    p = jnp.exp(e3 - m)
    s = jnp.sum(jnp.sum(p, axis=2, keepdims=True), axis=1, keepdims=True)
    s_ref[...] = (p / s).reshape(BLOCK_ROWS, PAIRS)


def _compute_scores(Wf, Bm):
    return pl.pallas_call(
        _scores_body,
        grid=(FLAT_ROWS // BLOCK_ROWS,),
        in_specs=[
            pl.BlockSpec((BLOCK_ROWS, FLAT_K), lambda i: (i, 0)),
            pl.BlockSpec((FLAT_K, PAIRS), lambda i: (0, 0)),
        ],
        out_specs=pl.BlockSpec((BLOCK_ROWS, PAIRS), lambda i: (i, 0)),
        out_shape=jax.ShapeDtypeStruct((FLAT_ROWS, PAIRS), jnp.float32),
    )(Wf, Bm)


def _make_sc_gather(B, D):
    info = plsc.get_sparse_core_info()
    NC, NS = info.num_cores, info.num_subcores
    NW = NC * NS
    b_per_w = B // NW
    mesh = plsc.VectorSubcoreMesh(core_axis_name="c", subcore_axis_name="s")

    @functools.partial(
        pl.kernel,
        mesh=mesh,
        out_type=jax.ShapeDtypeStruct((B, D), jnp.float32),
        scratch_types=[
            pltpu.VMEM((b_per_w,), jnp.int32),
            pltpu.VMEM((b_per_w, D), jnp.float32),
            pltpu.SemaphoreType.DMA,
        ],
    )
    def gather_k(table_hbm, idx_hbm, out_hbm, idx_v, rows_v, sem):
        wid = lax.axis_index("s") * NC + lax.axis_index("c")
        base = wid * b_per_w
        pltpu.sync_copy(idx_hbm.at[pl.ds(base, b_per_w)], idx_v)
        pltpu.async_copy(table_hbm.at[idx_v], rows_v, sem).wait()
        pltpu.sync_copy(rows_v, out_hbm.at[pl.ds(base, b_per_w)])

    return gather_k


def kernel(node_indices, context_vector, W):
    Wf = W.reshape(FLAT_ROWS, FLAT_K)
    Bm = (jnp.eye(PAIRS, dtype=jnp.float32)[:, None, :]
          * context_vector[None, :, None]).reshape(FLAT_K, PAIRS)
    scores_flat = _compute_scores(Wf, Bm.astype(jnp.bfloat16))
    scores = scores_flat.reshape(NUM_NODES, NUM_NODES)
    gather = _make_sc_gather(node_indices.shape[0], NUM_NODES)
    return gather(scores, node_indices.astype(jnp.int32))


# speedup vs baseline: 6.7455x; 6.1899x over previous
import functools

import jax
import jax.numpy as jnp
from jax import lax
from jax.experimental import pallas as pl
from jax.experimental.pallas import tpu as pltpu
from jax.experimental.pallas import tpu_sc as plsc

NUM_NODES = 1024
DIM = 64
BLOCK_M = 16


def _scores_body(w_ref, c_ref, s_ref):
    w = w_ref[...]
    c = c_ref[...]
    e = jnp.sum(w * c[None, :, :], axis=1)
    m = jnp.max(e, axis=-1, keepdims=True)
    p = jnp.exp(e - m)
    s_ref[...] = p / jnp.sum(p, axis=-1, keepdims=True)


def _compute_scores(Wt, c2):
    N = NUM_NODES
    return pl.pallas_call(
        _scores_body,
        grid=(N // BLOCK_M,),
        in_specs=[
            pl.BlockSpec((BLOCK_M, DIM, N), lambda i: (i, 0, 0)),
            pl.BlockSpec((DIM, 1), lambda i: (0, 0)),
        ],
        out_specs=pl.BlockSpec((BLOCK_M, N), lambda i: (i, 0)),
        out_shape=jax.ShapeDtypeStruct((N, N), jnp.float32),
    )(Wt, c2)


def _make_sc_gather(B, D):
    info = plsc.get_sparse_core_info()
    NC, NS = info.num_cores, info.num_subcores
    NW = NC * NS
    b_per_w = B // NW
    mesh = plsc.VectorSubcoreMesh(core_axis_name="c", subcore_axis_name="s")

    @functools.partial(
        pl.kernel,
        mesh=mesh,
        out_type=jax.ShapeDtypeStruct((B, D), jnp.float32),
        scratch_types=[
            pltpu.VMEM((b_per_w,), jnp.int32),
            pltpu.VMEM((b_per_w, D), jnp.float32),
            pltpu.SemaphoreType.DMA,
        ],
    )
    def gather_k(table_hbm, idx_hbm, out_hbm, idx_v, rows_v, sem):
        wid = lax.axis_index("s") * NC + lax.axis_index("c")
        base = wid * b_per_w
        pltpu.sync_copy(idx_hbm.at[pl.ds(base, b_per_w)], idx_v)
        pltpu.async_copy(table_hbm.at[idx_v], rows_v, sem).wait()
        pltpu.sync_copy(rows_v, out_hbm.at[pl.ds(base, b_per_w)])

    return gather_k


def kernel(node_indices, context_vector, W):
    Wt = W.transpose(0, 2, 1)
    c2 = context_vector.reshape(DIM, 1)
    scores = _compute_scores(Wt, c2)
    gather = _make_sc_gather(node_indices.shape[0], NUM_NODES)
    return gather(scores, node_indices.astype(jnp.int32))


# speedup vs baseline: 7.9841x; 1.1836x over previous
import functools

import jax
import jax.numpy as jnp
from jax import lax
from jax.experimental import pallas as pl
from jax.experimental.pallas import tpu as pltpu
from jax.experimental.pallas import tpu_sc as plsc

NUM_NODES = 1024
DIM = 64
BLOCK_M = 32


def _scores_body(w_ref, c_ref, s_ref):
    w = w_ref[...]
    c = c_ref[...]
    e = jnp.sum(w * c[None, :, :], axis=1)
    m = jnp.max(e, axis=-1, keepdims=True)
    p = jnp.exp(e - m)
    s_ref[...] = p / jnp.sum(p, axis=-1, keepdims=True)


def _compute_scores(Wt, c2):
    N = NUM_NODES
    return pl.pallas_call(
        _scores_body,
        grid=(N // BLOCK_M,),
        in_specs=[
            pl.BlockSpec((BLOCK_M, DIM, N), lambda i: (i, 0, 0)),
            pl.BlockSpec((DIM, 1), lambda i: (0, 0)),
        ],
        out_specs=pl.BlockSpec((BLOCK_M, N), lambda i: (i, 0)),
        out_shape=jax.ShapeDtypeStruct((N, N), jnp.float32),
    )(Wt, c2)


def _make_sc_gather(B, D):
    info = plsc.get_sparse_core_info()
    NC, NS = info.num_cores, info.num_subcores
    NW = NC * NS
    b_per_w = B // NW
    mesh = plsc.VectorSubcoreMesh(core_axis_name="c", subcore_axis_name="s")

    @functools.partial(
        pl.kernel,
        mesh=mesh,
        out_type=jax.ShapeDtypeStruct((B, D), jnp.float32),
        scratch_types=[
            pltpu.VMEM((b_per_w,), jnp.int32),
            pltpu.VMEM((b_per_w, D), jnp.float32),
            pltpu.SemaphoreType.DMA,
        ],
    )
    def gather_k(table_hbm, idx_hbm, out_hbm, idx_v, rows_v, sem):
        wid = lax.axis_index("s") * NC + lax.axis_index("c")
        base = wid * b_per_w
        pltpu.sync_copy(idx_hbm.at[pl.ds(base, b_per_w)], idx_v)
        pltpu.async_copy(table_hbm.at[idx_v], rows_v, sem).wait()
        pltpu.sync_copy(rows_v, out_hbm.at[pl.ds(base, b_per_w)])

    return gather_k


def kernel(node_indices, context_vector, W):
    Wt = W.transpose(0, 2, 1)
    c2 = context_vector.reshape(DIM, 1)
    scores = _compute_scores(Wt, c2)
    gather = _make_sc_gather(node_indices.shape[0], NUM_NODES)
    return gather(scores, node_indices.astype(jnp.int32))


# speedup vs baseline: 8.0428x; 1.0073x over previous
import functools

import jax
import jax.numpy as jnp
from jax import lax
from jax.experimental import pallas as pl
from jax.experimental.pallas import tpu as pltpu
from jax.experimental.pallas import tpu_sc as plsc

NUM_NODES = 1024
DIM = 64
BLOCK_M = 64


def _scores_body(w_ref, c_ref, s_ref):
    w = w_ref[...]
    c = c_ref[...]
    e = jnp.sum(w * c[None, :, :], axis=1)
    m = jnp.max(e, axis=-1, keepdims=True)
    p = jnp.exp(e - m)
    s_ref[...] = p / jnp.sum(p, axis=-1, keepdims=True)


def _compute_scores(Wt, c2):
    N = NUM_NODES
    return pl.pallas_call(
        _scores_body,
        grid=(N // BLOCK_M,),
        in_specs=[
            pl.BlockSpec((BLOCK_M, DIM, N), lambda i: (i, 0, 0)),
            pl.BlockSpec((DIM, 1), lambda i: (0, 0)),
        ],
        out_specs=pl.BlockSpec((BLOCK_M, N), lambda i: (i, 0)),
        out_shape=jax.ShapeDtypeStruct((N, N), jnp.float32),
    )(Wt, c2)


def _make_sc_gather(B, D):
    info = plsc.get_sparse_core_info()
    NC, NS = info.num_cores, info.num_subcores
    NW = NC * NS
    b_per_w = B // NW
    mesh = plsc.VectorSubcoreMesh(core_axis_name="c", subcore_axis_name="s")

    @functools.partial(
        pl.kernel,
        mesh=mesh,
        out_type=jax.ShapeDtypeStruct((B, D), jnp.float32),
        scratch_types=[
            pltpu.VMEM((b_per_w,), jnp.int32),
            pltpu.VMEM((b_per_w, D), jnp.float32),
            pltpu.SemaphoreType.DMA,
        ],
    )
    def gather_k(table_hbm, idx_hbm, out_hbm, idx_v, rows_v, sem):
        wid = lax.axis_index("s") * NC + lax.axis_index("c")
        base = wid * b_per_w
        pltpu.sync_copy(idx_hbm.at[pl.ds(base, b_per_w)], idx_v)
        pltpu.async_copy(table_hbm.at[idx_v], rows_v, sem).wait()
        pltpu.sync_copy(rows_v, out_hbm.at[pl.ds(base, b_per_w)])

    return gather_k


def kernel(node_indices, context_vector, W):
    Wt = W.transpose(0, 2, 1)
    c2 = context_vector.reshape(DIM, 1)
    scores = _compute_scores(Wt, c2)
    gather = _make_sc_gather(node_indices.shape[0], NUM_NODES)
    return gather(scores, node_indices.astype(jnp.int32))
